# baseline (speedup 1.0000x reference)
import jax
import jax.numpy as jnp
from jax import lax
from jax.experimental import pallas as pl
from jax.experimental.pallas import tpu as pltpu
from jax.experimental.pallas import tpu_sc as plsc

NTOKEN = 1000000
EMB_DIM = 64
BATCH = 16384
HIST_LEN = 50

NC = 2
NS = 16
NW = NC * NS

NB = BATCH // NW
G = 16
N_GROUPS = NB // G
T = N_GROUPS // 2


def _body(table_hbm, x_hbm, out_hbm, slab, buf0, buf1, gsem0, gsem1,
          ssem0, ssem1):
  wid = lax.axis_index("s") * NC + lax.axis_index("c")
  b0 = wid * NB

  pltpu.sync_copy(x_hbm.at[pl.ds(b0, NB), :], slab)

  def issue_gathers(g, buf, sem):
    for i in range(G):
      pltpu.async_copy(table_hbm.at[slab.at[g * G + i]], buf.at[i], sem)

  def wait_gathers(buf, sem):
    for i in range(G):
      pltpu.make_async_copy(table_hbm.at[slab.at[i]], buf.at[i], sem).wait()

  def issue_store(g, buf, sem):
    pltpu.async_copy(buf, out_hbm.at[pl.ds(b0 + g * G, G), :, :], sem)

  def wait_store(buf, sem):
    pltpu.make_async_copy(buf, out_hbm.at[pl.ds(b0, G), :, :], sem).wait()

  issue_gathers(0, buf0, gsem0)

  @pl.loop(0, T)
  def _(t):
    a = 2 * t

    @pl.when(t > 0)
    def _():
      wait_store(buf1, ssem1)
    issue_gathers(a + 1, buf1, gsem1)

    wait_gathers(buf0, gsem0)
    issue_store(a, buf0, ssem0)

    @pl.when(t < T - 1)
    def _():
      wait_store(buf0, ssem0)
      issue_gathers(a + 2, buf0, gsem0)

    wait_gathers(buf1, gsem1)
    issue_store(a + 1, buf1, ssem1)

  wait_store(buf0, ssem0)
  wait_store(buf1, ssem1)


@jax.jit
def _lookup(x2d, emb_weight):
  mesh = plsc.VectorSubcoreMesh(
      core_axis_name="c", subcore_axis_name="s", num_cores=NC,
      num_subcores=NS)
  scratch = [
      pltpu.VMEM((NB, HIST_LEN), jnp.int32),
      pltpu.VMEM((G, HIST_LEN, EMB_DIM), jnp.float32),
      pltpu.VMEM((G, HIST_LEN, EMB_DIM), jnp.float32),
      pltpu.SemaphoreType.DMA,
      pltpu.SemaphoreType.DMA,
      pltpu.SemaphoreType.DMA,
      pltpu.SemaphoreType.DMA,
  ]
  return pl.kernel(
      _body,
      out_type=jax.ShapeDtypeStruct((BATCH, HIST_LEN, EMB_DIM), jnp.float32),
      mesh=mesh,
      scratch_types=scratch,
      compiler_params=pltpu.CompilerParams(use_tc_tiling_on_sc=False),
  )(emb_weight, x2d)


def kernel(x, emb_weight):
  return _lookup(x.astype(jnp.int32), emb_weight)
